# baseline (speedup 1.0000x reference)
import functools
import jax
import jax.numpy as jnp
from jax import lax
from jax.experimental import pallas as pl
from jax.experimental.pallas import tpu as pltpu
from jax.experimental.pallas import tpu_sc as plsc

B, L, D = 1024, 200, 64
NSEG = 3
NW = 32
ROWS = B * L
ROWS_PER_W = ROWS // NW
CHUNK = 400
NCHUNK = ROWS_PER_W // CHUNK
PIECE = 80
NPIECE = CHUNK // PIECE
LANES = 16
NGRP = CHUNK // LANES


def _sc_body(xf, segf, table, seg_table, pos, out,
             idxall, segall, rowbuf0, rowbuf1, posbuf, segtbuf, addbuf,
             gsem, osem):
    wid = lax.axis_index("s") * 2 + lax.axis_index("c")
    wbase = wid * ROWS_PER_W

    pltpu.sync_copy(xf.at[pl.ds(wbase, ROWS_PER_W)], idxall)
    pltpu.sync_copy(segf.at[pl.ds(wbase, ROWS_PER_W)], segall)

    def fire_gathers(c, buf):
        for p in range(NPIECE):
            pltpu.async_copy(
                table.at[idxall.at[pl.ds(c * CHUNK + p * PIECE, PIECE)]],
                buf.at[pl.ds(p * PIECE, PIECE)], gsem)

    def wait_gathers(c, buf):
        for p in range(NPIECE):
            pltpu.make_async_copy(
                table.at[idxall.at[pl.ds(c * CHUNK + p * PIECE, PIECE)]],
                buf.at[pl.ds(p * PIECE, PIECE)], gsem).wait()

    def fire_out(c, buf):
        pltpu.async_copy(buf, out.at[pl.ds(wbase + c * CHUNK, CHUNK)], osem)

    def wait_out(c, buf):
        pltpu.make_async_copy(
            buf, out.at[pl.ds(wbase + c * CHUNK, CHUNK)], osem).wait()

    fire_gathers(0, rowbuf0)

    pltpu.sync_copy(pos, posbuf)
    pltpu.sync_copy(seg_table, segtbuf)
    segrows = [[segtbuf[s, pl.ds(LANES * k, LANES)] for k in range(D // LANES)]
               for s in range(NSEG)]

    def addloop(l, carry):
        for k in range(D // LANES):
            pk = posbuf[l, pl.ds(LANES * k, LANES)]
            for s in range(NSEG):
                addbuf[pl.ds(s * (L * D) + l * D + LANES * k, LANES)] = (
                    pk + segrows[s][k])
        return carry

    lax.fori_loop(0, L, addloop, 0)

    iota = lax.iota(jnp.int32, LANES)

    def compute(c, buf):
        coff = c * CHUNK

        def grp_body(g, carry2):
            segv = segall[pl.ds(coff + g * LANES, LANES)]
            lposv = lax.rem(coff + g * LANES + iota, L)
            fbv = segv * (L * D) + lposv * D
            rowv = g * LANES + iota
            for d in range(D):
                gv = plsc.load_gather(addbuf, [fbv + d])
                colv = jnp.full((LANES,), d, dtype=jnp.int32)
                plsc.addupdate_scatter(buf, [rowv, colv], gv)
            return carry2

        lax.fori_loop(0, NGRP, grp_body, 0)

    def pair_body(t, carry):
        c0 = 2 * t

        @pl.when(t >= 1)
        def _():
            wait_out(c0 - 1, rowbuf1)

        fire_gathers(c0 + 1, rowbuf1)
        wait_gathers(c0, rowbuf0)
        compute(c0, rowbuf0)
        fire_out(c0, rowbuf0)

        c1 = c0 + 1
        wait_out(c0, rowbuf0)

        @pl.when(t < NCHUNK // 2 - 1)
        def _():
            fire_gathers(c1 + 1, rowbuf0)

        wait_gathers(c1, rowbuf1)
        compute(c1, rowbuf1)
        fire_out(c1, rowbuf1)
        return carry

    lax.fori_loop(0, NCHUNK // 2, pair_body, 0)
    wait_out(NCHUNK - 1, rowbuf1)


_sc_kernel = functools.partial(
    pl.kernel,
    mesh=plsc.VectorSubcoreMesh(core_axis_name="c", subcore_axis_name="s"),
    out_type=jax.ShapeDtypeStruct((ROWS, D), jnp.float32),
    scratch_types=[
        pltpu.VMEM((ROWS_PER_W,), jnp.int32),
        pltpu.VMEM((ROWS_PER_W,), jnp.int32),
        pltpu.VMEM((CHUNK, D), jnp.float32),
        pltpu.VMEM((CHUNK, D), jnp.float32),
        pltpu.VMEM((L, D), jnp.float32),
        pltpu.VMEM((NSEG, D), jnp.float32),
        pltpu.VMEM((NSEG * L * D,), jnp.float32),
        pltpu.SemaphoreType.DMA,
        pltpu.SemaphoreType.DMA,
    ],
    compiler_params=pltpu.CompilerParams(
        needs_layout_passes=False, use_tc_tiling_on_sc=False),
)(_sc_body)


def kernel(x, segment_info, table, seg_table, pos):
    xf = x.astype(jnp.int32).reshape(ROWS)
    segf = segment_info.astype(jnp.int32).reshape(ROWS)
    pos_l = pos[:L]
    out = _sc_kernel(xf, segf, table, seg_table, pos_l)
    return out.reshape(B, L, D)


# speedup vs baseline: 1.5085x; 1.5085x over previous
import functools
import jax
import jax.numpy as jnp
from jax import lax
from jax.experimental import pallas as pl
from jax.experimental.pallas import tpu as pltpu
from jax.experimental.pallas import tpu_sc as plsc

B, L, D = 1024, 200, 64
NSEG = 3
NW = 32
ROWS = B * L
ROWS_PER_W = ROWS // NW
CHUNK = 400
NCHUNK = ROWS_PER_W // CHUNK
PIECE = 80
NPIECE = CHUNK // PIECE
LANES = 16
NGRP = CHUNK // LANES


def _sc_body(xf, segf, table, seg_table, pos, out,
             idxall, segall, rowbuf0, rowbuf1, posbuf,
             segtbuf, addbuf, gsem, osem):
    wid = lax.axis_index("s") * 2 + lax.axis_index("c")
    wbase = wid * ROWS_PER_W

    pltpu.sync_copy(xf.at[pl.ds(wbase, ROWS_PER_W)], idxall)
    pltpu.sync_copy(segf.at[pl.ds(wbase, ROWS_PER_W)], segall)

    def fire_gathers(c, buf):
        for p in range(NPIECE):
            pltpu.async_copy(
                table.at[idxall.at[pl.ds(c * CHUNK + p * PIECE, PIECE)]],
                buf.at[pl.ds(p * PIECE, PIECE)], gsem)

    def wait_gathers(c, buf):
        for p in range(NPIECE):
            pltpu.make_async_copy(
                table.at[idxall.at[pl.ds(c * CHUNK + p * PIECE, PIECE)]],
                buf.at[pl.ds(p * PIECE, PIECE)], gsem).wait()

    def fire_out(c, buf):
        pltpu.async_copy(buf, out.at[pl.ds(wbase + c * CHUNK, CHUNK)], osem)

    def wait_out(c, buf):
        pltpu.make_async_copy(
            buf, out.at[pl.ds(wbase + c * CHUNK, CHUNK)], osem).wait()

    fire_gathers(0, rowbuf0)

    pltpu.sync_copy(pos, posbuf)
    pltpu.sync_copy(seg_table, segtbuf)
    segrows = [[segtbuf[s, pl.ds(LANES * k, LANES)] for k in range(D // LANES)]
               for s in range(NSEG)]

    def addloop(l, carry):
        for k in range(D // LANES):
            pk = posbuf[l, pl.ds(LANES * k, LANES)]
            for s in range(NSEG):
                addbuf[pl.ds(s * (L * D) + l * D + LANES * k, LANES)] = (
                    pk + segrows[s][k])
        return carry

    lax.fori_loop(0, L, addloop, 0)

    iota = lax.iota(jnp.int32, LANES)
    iotas = [iota + LANES * k for k in range(D // LANES)]
    jfulls = [jnp.full((LANES,), j, dtype=jnp.int32) for j in range(LANES)]

    def compute(c, buf):
        coff = c * CHUNK

        def grp_body(g, carry2):
            segv = segall[pl.ds(coff + g * LANES, LANES)]
            lposv = lax.rem(g * LANES + iota, L)
            fbv = segv * (L * D) + lposv * D
            for j in range(LANES):
                fbs = fbv.at[jfulls[j]].get(mode="promise_in_bounds")
                r = g * LANES + j
                for k in range(D // LANES):
                    av = plsc.load_gather(addbuf, [fbs + iotas[k]])
                    plsc.addupdate(buf.at[r, pl.ds(LANES * k, LANES)], av)
            return carry2

        lax.fori_loop(0, NGRP, grp_body, 0)

    def pair_body(t, carry):
        c0 = 2 * t

        @pl.when(t >= 1)
        def _():
            wait_out(c0 - 1, rowbuf1)

        fire_gathers(c0 + 1, rowbuf1)
        wait_gathers(c0, rowbuf0)
        compute(c0, rowbuf0)
        fire_out(c0, rowbuf0)

        c1 = c0 + 1
        wait_out(c0, rowbuf0)

        @pl.when(t < NCHUNK // 2 - 1)
        def _():
            fire_gathers(c1 + 1, rowbuf0)

        wait_gathers(c1, rowbuf1)
        compute(c1, rowbuf1)
        fire_out(c1, rowbuf1)
        return carry

    lax.fori_loop(0, NCHUNK // 2, pair_body, 0)
    wait_out(NCHUNK - 1, rowbuf1)


_sc_kernel = functools.partial(
    pl.kernel,
    mesh=plsc.VectorSubcoreMesh(core_axis_name="c", subcore_axis_name="s"),
    out_type=jax.ShapeDtypeStruct((ROWS, D), jnp.float32),
    scratch_types=[
        pltpu.VMEM((ROWS_PER_W,), jnp.int32),
        pltpu.VMEM((ROWS_PER_W,), jnp.int32),
        pltpu.VMEM((CHUNK, D), jnp.float32),
        pltpu.VMEM((CHUNK, D), jnp.float32),
        pltpu.VMEM((L, D), jnp.float32),
        pltpu.VMEM((NSEG, D), jnp.float32),
        pltpu.VMEM((NSEG * L * D,), jnp.float32),
        pltpu.SemaphoreType.DMA,
        pltpu.SemaphoreType.DMA,
    ],
    compiler_params=pltpu.CompilerParams(
        needs_layout_passes=False, use_tc_tiling_on_sc=False),
)(_sc_body)


def kernel(x, segment_info, table, seg_table, pos):
    xf = x.astype(jnp.int32).reshape(ROWS)
    segf = segment_info.astype(jnp.int32).reshape(ROWS)
    pos_l = pos[:L]
    out = _sc_kernel(xf, segf, table, seg_table, pos_l)
    return out.reshape(B, L, D)
